# baseline (speedup 1.0000x reference)
import functools

import jax
import jax.numpy as jnp
from jax import lax
from jax.experimental import pallas as pl
from jax.experimental.pallas import tpu as pltpu
from jax.experimental.pallas import tpu_sc as plsc

B, T, D = 4096, 200, 64
NC, NS = 2, 16
NW = NC * NS
BL = B // NW
STEPS = T
MAIN = (STEPS - 2) // 3

WP = 137
_ROWS = pltpu.VMEM((BL, D), jnp.bfloat16)
_IDX = pltpu.VMEM((BL,), jnp.int32)
_WB = pltpu.VMEM((D, WP), jnp.float32)
_SEM = pltpu.SemaphoreType.DMA


@functools.partial(
    pl.kernel,
    out_type=jax.ShapeDtypeStruct((T, 8, NW, 8, 128), jnp.float32),
    mesh=plsc.VectorSubcoreMesh(core_axis_name="c", subcore_axis_name="s"),
    scratch_types=[
        _IDX, _IDX, _IDX, _IDX, _IDX, _IDX, _IDX, _IDX, _IDX,
        _ROWS, _ROWS, _ROWS, _ROWS, _ROWS, _ROWS, _ROWS, _ROWS, _ROWS,
        _WB, _WB, _WB,
        _SEM, _SEM, _SEM, _SEM, _SEM, _SEM, _SEM, _SEM, _SEM,
    ],
    compiler_params=pltpu.CompilerParams(
        use_tc_tiling_on_sc=False, needs_layout_passes=False),
)
def _encode(i1, i2, i3, t1, t2, t3, out,
            xa1, xa2, xa3, xb1, xb2, xb3, xc1, xc2, xc3,
            ra1, ra2, ra3, rb1, rb2, rb3, rc1, rc2, rc3,
            wba, wbb, wbc,
            gsa, gsb, gsc, osa, osb, osc, ia, ib, ic):
    wid = lax.axis_index("s") * NC + lax.axis_index("c")
    b0 = wid * BL
    iota = lax.iota(jnp.int32, 16)

    def issue_idx(s, x1, x2, x3, sem):
        o = s * B + b0
        pltpu.async_copy(i1.at[pl.ds(o, BL)], x1, sem)
        pltpu.async_copy(i2.at[pl.ds(o, BL)], x2, sem)
        pltpu.async_copy(i3.at[pl.ds(o, BL)], x3, sem)

    def wait_idx(x1, x2, x3, sem):
        pltpu.make_async_copy(i1.at[pl.ds(0, BL)], x1, sem).wait()
        pltpu.make_async_copy(i2.at[pl.ds(0, BL)], x2, sem).wait()
        pltpu.make_async_copy(i3.at[pl.ds(0, BL)], x3, sem).wait()

    def issue_g(x1, x2, x3, r1, r2, r3, sem):
        pltpu.async_copy(t1.at[x1], r1, sem)
        pltpu.async_copy(t2.at[x2], r2, sem)
        pltpu.async_copy(t3.at[x3], r3, sem)

    def wait_g(x1, x2, x3, r1, r2, r3, sem):
        pltpu.make_async_copy(t1.at[x1], r1, sem).wait()
        pltpu.make_async_copy(t2.at[x2], r2, sem).wait()
        pltpu.make_async_copy(t3.at[x3], r3, sem).wait()

    deven = [iota * 2 + h * 32 for h in range(2)]
    dodd = [iota * 2 + h * 32 + 1 for h in range(2)]
    himask = jnp.full((16,), -65536, jnp.int32)

    def add_tr(r1, r2, r3, wb):
        @plsc.parallel_loop(0, BL, unroll=16)
        def body(bl):
            colv = jnp.broadcast_to(bl, (16,))
            for h in range(2):
                sl = pl.ds(h * 32, 32)
                w1 = plsc.bitcast(r1[bl, sl], jnp.int32)
                w2 = plsc.bitcast(r2[bl, sl], jnp.int32)
                w3 = plsc.bitcast(r3[bl, sl], jnp.int32)
                ve = (plsc.bitcast(w1 << 16, jnp.float32)
                      + plsc.bitcast(w2 << 16, jnp.float32)
                      + plsc.bitcast(w3 << 16, jnp.float32))
                vo = (plsc.bitcast(w1 & himask, jnp.float32)
                      + plsc.bitcast(w2 & himask, jnp.float32)
                      + plsc.bitcast(w3 & himask, jnp.float32))
                plsc.store_scatter(wb, [deven[h], colv], ve)
                plsc.store_scatter(wb, [dodd[h], colv], vo)

    def issue_w(s, wb, sem):
        for dt in range(8):
            pltpu.async_copy(wb.at[pl.ds(dt * 8, 8), pl.ds(0, 128)],
                             out.at[s, dt, wid], sem)

    def wait_w(wb, sem):
        for dt in range(8):
            pltpu.make_async_copy(wb.at[pl.ds(0, 8), pl.ds(0, 128)],
                                  out.at[0, 0, wid], sem).wait()

    issue_idx(0, xa1, xa2, xa3, ia)
    wait_idx(xa1, xa2, xa3, ia)
    issue_g(xa1, xa2, xa3, ra1, ra2, ra3, gsa)
    issue_idx(1, xb1, xb2, xb3, ib)

    def ring(k, c):
        s0 = k * 3

        @pl.when(k > 0)
        def _():
            wait_w(wbb, osb)
        wait_idx(xb1, xb2, xb3, ib)
        issue_g(xb1, xb2, xb3, rb1, rb2, rb3, gsb)
        issue_idx(s0 + 2, xc1, xc2, xc3, ic)
        wait_g(xa1, xa2, xa3, ra1, ra2, ra3, gsa)
        add_tr(ra1, ra2, ra3, wba)
        issue_w(s0, wba, osa)

        @pl.when(k > 0)
        def _():
            wait_w(wbc, osc)
        wait_idx(xc1, xc2, xc3, ic)
        issue_g(xc1, xc2, xc3, rc1, rc2, rc3, gsc)
        issue_idx(s0 + 3, xa1, xa2, xa3, ia)
        wait_g(xb1, xb2, xb3, rb1, rb2, rb3, gsb)
        add_tr(rb1, rb2, rb3, wbb)
        issue_w(s0 + 1, wbb, osb)

        wait_w(wba, osa)
        wait_idx(xa1, xa2, xa3, ia)
        issue_g(xa1, xa2, xa3, ra1, ra2, ra3, gsa)
        issue_idx(s0 + 4, xb1, xb2, xb3, ib)
        wait_g(xc1, xc2, xc3, rc1, rc2, rc3, gsc)
        add_tr(rc1, rc2, rc3, wbc)
        issue_w(s0 + 2, wbc, osc)
        return c

    lax.fori_loop(0, MAIN, ring, 0)

    wait_w(wbb, osb)
    wait_idx(xb1, xb2, xb3, ib)
    issue_g(xb1, xb2, xb3, rb1, rb2, rb3, gsb)
    wait_g(xa1, xa2, xa3, ra1, ra2, ra3, gsa)
    add_tr(ra1, ra2, ra3, wba)
    issue_w(STEPS - 2, wba, osa)
    wait_w(wbc, osc)
    wait_g(xb1, xb2, xb3, rb1, rb2, rb3, gsb)
    add_tr(rb1, rb2, rb3, wbb)
    issue_w(STEPS - 1, wbb, osb)
    wait_w(wba, osa)
    wait_w(wbb, osb)


def kernel(mcc_code, tr_type, country, seq_lens, emb_mcc, emb_tr, emb_cty):
    t1 = emb_mcc.at[0].set(0.0).astype(jnp.bfloat16)
    t2 = emb_tr.at[0].set(0.0).astype(jnp.bfloat16)
    t3 = emb_cty.at[0].set(0.0).astype(jnp.bfloat16)
    i1 = mcc_code.T.reshape(T * B)
    i2 = tr_type.T.reshape(T * B)
    i3 = country.T.reshape(T * B)
    out5 = _encode(i1, i2, i3, t1, t2, t3)
    x = out5.transpose(2, 4, 0, 1, 3)
    return x.reshape(B, T, D)


# speedup vs baseline: 2.0927x; 2.0927x over previous
import functools

import jax
import jax.numpy as jnp
from jax import lax
from jax.experimental import pallas as pl
from jax.experimental.pallas import tpu as pltpu
from jax.experimental.pallas import tpu_sc as plsc

B, T, D = 4096, 200, 64
NC, NS = 2, 16
NW = NC * NS
BL = B // NW
STEPS = T
MAIN = (STEPS - 2) // 3

WP = 137
_ROWS = pltpu.VMEM((BL, D), jnp.float32)
_IDX = pltpu.VMEM((BL,), jnp.int32)
_WB = pltpu.VMEM((D, WP), jnp.float32)
_SEM = pltpu.SemaphoreType.DMA


@functools.partial(
    pl.kernel,
    out_type=jax.ShapeDtypeStruct((T, 8, NW, 8, 128), jnp.float32),
    mesh=plsc.VectorSubcoreMesh(core_axis_name="c", subcore_axis_name="s"),
    scratch_types=[
        _IDX, _IDX, _IDX, _IDX, _IDX, _IDX, _IDX, _IDX, _IDX,
        _ROWS, _ROWS, _ROWS, _ROWS, _ROWS, _ROWS, _ROWS, _ROWS, _ROWS,
        _WB, _WB, _WB,
        _SEM, _SEM, _SEM, _SEM, _SEM, _SEM, _SEM, _SEM, _SEM,
    ],
    compiler_params=pltpu.CompilerParams(
        use_tc_tiling_on_sc=False, needs_layout_passes=False),
)
def _encode(i1, i2, i3, t1, t2, t3, out,
            xa1, xa2, xa3, xb1, xb2, xb3, xc1, xc2, xc3,
            ra1, ra2, ra3, rb1, rb2, rb3, rc1, rc2, rc3,
            wba, wbb, wbc,
            gsa, gsb, gsc, osa, osb, osc, ia, ib, ic):
    wid = lax.axis_index("s") * NC + lax.axis_index("c")
    b0 = wid * BL
    iota = lax.iota(jnp.int32, 16)

    def issue_idx(s, x1, x2, x3, sem):
        o = s * B + b0
        pltpu.async_copy(i1.at[pl.ds(o, BL)], x1, sem)
        pltpu.async_copy(i2.at[pl.ds(o, BL)], x2, sem)
        pltpu.async_copy(i3.at[pl.ds(o, BL)], x3, sem)

    def wait_idx(x1, x2, x3, sem):
        pltpu.make_async_copy(i1.at[pl.ds(0, BL)], x1, sem).wait()
        pltpu.make_async_copy(i2.at[pl.ds(0, BL)], x2, sem).wait()
        pltpu.make_async_copy(i3.at[pl.ds(0, BL)], x3, sem).wait()

    def issue_g(x1, x2, x3, r1, r2, r3, sem):
        pltpu.async_copy(t1.at[x1], r1, sem)
        pltpu.async_copy(t2.at[x2], r2, sem)
        pltpu.async_copy(t3.at[x3], r3, sem)

    def wait_g(x1, x2, x3, r1, r2, r3, sem):
        pltpu.make_async_copy(t1.at[x1], r1, sem).wait()
        pltpu.make_async_copy(t2.at[x2], r2, sem).wait()
        pltpu.make_async_copy(t3.at[x3], r3, sem).wait()

    d16 = [iota + dg * 16 for dg in range(4)]

    def add_tr(r1, r2, r3, wb):
        @plsc.parallel_loop(0, BL, unroll=16)
        def body(bl):
            colv = jnp.broadcast_to(bl, (16,))
            for dg in range(4):
                sl = pl.ds(dg * 16, 16)
                v = r1[bl, sl] + r2[bl, sl] + r3[bl, sl]
                plsc.store_scatter(wb, [d16[dg], colv], v)

    def issue_w(s, wb, sem):
        for dt in range(8):
            pltpu.async_copy(wb.at[pl.ds(dt * 8, 8), pl.ds(0, 128)],
                             out.at[s, dt, wid], sem)

    def wait_w(wb, sem):
        for dt in range(8):
            pltpu.make_async_copy(wb.at[pl.ds(0, 8), pl.ds(0, 128)],
                                  out.at[0, 0, wid], sem).wait()

    issue_idx(0, xa1, xa2, xa3, ia)
    wait_idx(xa1, xa2, xa3, ia)
    issue_g(xa1, xa2, xa3, ra1, ra2, ra3, gsa)
    issue_idx(1, xb1, xb2, xb3, ib)

    def ring(k, c):
        s0 = k * 3

        @pl.when(k > 0)
        def _():
            wait_w(wbb, osb)
        wait_idx(xb1, xb2, xb3, ib)
        issue_g(xb1, xb2, xb3, rb1, rb2, rb3, gsb)
        issue_idx(s0 + 2, xc1, xc2, xc3, ic)
        wait_g(xa1, xa2, xa3, ra1, ra2, ra3, gsa)
        add_tr(ra1, ra2, ra3, wba)
        issue_w(s0, wba, osa)

        @pl.when(k > 0)
        def _():
            wait_w(wbc, osc)
        wait_idx(xc1, xc2, xc3, ic)
        issue_g(xc1, xc2, xc3, rc1, rc2, rc3, gsc)
        issue_idx(s0 + 3, xa1, xa2, xa3, ia)
        wait_g(xb1, xb2, xb3, rb1, rb2, rb3, gsb)
        add_tr(rb1, rb2, rb3, wbb)
        issue_w(s0 + 1, wbb, osb)

        wait_w(wba, osa)
        wait_idx(xa1, xa2, xa3, ia)
        issue_g(xa1, xa2, xa3, ra1, ra2, ra3, gsa)
        issue_idx(s0 + 4, xb1, xb2, xb3, ib)
        wait_g(xc1, xc2, xc3, rc1, rc2, rc3, gsc)
        add_tr(rc1, rc2, rc3, wbc)
        issue_w(s0 + 2, wbc, osc)
        return c

    lax.fori_loop(0, MAIN, ring, 0)

    wait_w(wbb, osb)
    wait_idx(xb1, xb2, xb3, ib)
    issue_g(xb1, xb2, xb3, rb1, rb2, rb3, gsb)
    wait_g(xa1, xa2, xa3, ra1, ra2, ra3, gsa)
    add_tr(ra1, ra2, ra3, wba)
    issue_w(STEPS - 2, wba, osa)
    wait_w(wbc, osc)
    wait_g(xb1, xb2, xb3, rb1, rb2, rb3, gsb)
    add_tr(rb1, rb2, rb3, wbb)
    issue_w(STEPS - 1, wbb, osb)
    wait_w(wba, osa)
    wait_w(wbb, osb)


def kernel(mcc_code, tr_type, country, seq_lens, emb_mcc, emb_tr, emb_cty):
    t1 = emb_mcc.at[0].set(0.0)
    t2 = emb_tr.at[0].set(0.0)
    t3 = emb_cty.at[0].set(0.0)
    i1 = mcc_code.T.reshape(T * B)
    i2 = tr_type.T.reshape(T * B)
    i3 = country.T.reshape(T * B)
    out5 = _encode(i1, i2, i3, t1, t2, t3)
    x = out5.transpose(2, 4, 0, 1, 3)
    return x.reshape(B, T, D)
